# baseline (speedup 1.0000x reference)
import functools

import jax
import jax.numpy as jnp
from jax import lax
from jax.experimental import pallas as pl
from jax.experimental.pallas import tpu as pltpu
from jax.experimental.pallas import tpu_sc as plsc

BATCH = 16384
OUT_DIM = 64
ID_DIM = 32
TEXT_DIM = 128

_NC = 2
_NS = 16
_NW = _NC * _NS
_BPW = BATCH // _NW


def _sc_gather(uids_hbm, iids_hbm, uemb_hbm, iemb_hbm, u_out, i_out,
               uidx_v, iidx_v, urows_v, irows_v, sem_u, sem_i):
    wid = lax.axis_index("s") * _NC + lax.axis_index("c")
    base = wid * _BPW
    pltpu.sync_copy(uids_hbm.at[pl.ds(base, _BPW)], uidx_v)
    pltpu.sync_copy(iids_hbm.at[pl.ds(base, _BPW)], iidx_v)
    cp_u = pltpu.async_copy(uemb_hbm.at[uidx_v], urows_v, sem_u)
    cp_i = pltpu.async_copy(iemb_hbm.at[iidx_v], irows_v, sem_i)
    cp_u.wait()
    cp_i.wait()
    pltpu.sync_copy(urows_v, u_out.at[pl.ds(base, _BPW)])
    pltpu.sync_copy(irows_v, i_out.at[pl.ds(base, _BPW)])


@functools.cache
def _gather_call():
    return pl.kernel(
        _sc_gather,
        mesh=plsc.VectorSubcoreMesh(core_axis_name="c", subcore_axis_name="s"),
        out_type=(
            jax.ShapeDtypeStruct((BATCH, OUT_DIM), jnp.float32),
            jax.ShapeDtypeStruct((BATCH, ID_DIM), jnp.float32),
        ),
        scratch_types=[
            pltpu.VMEM((_BPW,), jnp.int32),
            pltpu.VMEM((_BPW,), jnp.int32),
            pltpu.VMEM((_BPW, OUT_DIM), jnp.float32),
            pltpu.VMEM((_BPW, ID_DIM), jnp.float32),
            pltpu.SemaphoreType.DMA,
            pltpu.SemaphoreType.DMA,
        ],
        compiler_params=pltpu.CompilerParams(use_tc_tiling_on_sc=False),
    )


_TC_ROWS = 512
_N_BLOCKS = BATCH // _TC_ROWS


def _tc_combine(x_ref, u_ref, id_ref, w_ref, b_ref, out_ref):
    t = jnp.dot(x_ref[...], w_ref[...], preferred_element_type=jnp.float32)
    t = t + b_ref[...]
    s = jnp.sum(u_ref[:, :ID_DIM] * id_ref[...], axis=1)
    s = s + jnp.sum(u_ref[:, ID_DIM:] * t, axis=1)
    out_ref[...] = jax.nn.sigmoid(s)


def _combine(x, u_gath, i_gath, W_text, b2):
    return pl.pallas_call(
        _tc_combine,
        grid=(_N_BLOCKS,),
        in_specs=[
            pl.BlockSpec((_TC_ROWS, TEXT_DIM), lambda i: (i, 0)),
            pl.BlockSpec((_TC_ROWS, OUT_DIM), lambda i: (i, 0)),
            pl.BlockSpec((_TC_ROWS, ID_DIM), lambda i: (i, 0)),
            pl.BlockSpec((TEXT_DIM, ID_DIM), lambda i: (0, 0)),
            pl.BlockSpec((1, ID_DIM), lambda i: (0, 0)),
        ],
        out_specs=pl.BlockSpec((_TC_ROWS,), lambda i: (i,)),
        out_shape=jax.ShapeDtypeStruct((BATCH,), jnp.float32),
    )(x, u_gath, i_gath, W_text, b2)


def kernel(user_ids, item_ids, item_text_feats, user_emb, item_id_emb,
           W_text, b_text):
    u_gath, i_gath = _gather_call()(user_ids, item_ids, user_emb, item_id_emb)
    return _combine(item_text_feats, u_gath, i_gath, W_text,
                    b_text.reshape(1, ID_DIM))
